# baseline (speedup 1.0000x reference)
import functools

import jax
import jax.numpy as jnp
from jax import lax
from jax.experimental import pallas as pl
from jax.experimental.pallas import tpu as pltpu
from jax.experimental.pallas import tpu_sc as plsc

EPS = 1e-5

_info = plsc.get_sparse_core_info()
NC, NS, NL = _info.num_cores, _info.num_subcores, _info.num_lanes
NW = NC * NS


def _make_pool_kernel(B, S, D, DP, CHUNK, REAL):
    B_PER_W = B // NW
    CPR = S // REAL
    CPW = B_PER_W * CPR
    G = D // NL
    inv_s = 1.0 / S
    NBUF = 3
    UNROLL = 4

    mesh = plsc.VectorSubcoreMesh(core_axis_name="c", subcore_axis_name="s")

    @functools.partial(
        pl.kernel,
        mesh=mesh,
        out_type=jax.ShapeDtypeStruct((B, D), jnp.float32),
        scratch_types=(
            [pltpu.VMEM((CPW, CHUNK), jnp.int32)]
            + [pltpu.VMEM((CHUNK, DP), jnp.float32) for _ in range(NBUF)]
            + [pltpu.VMEM((B_PER_W, D), jnp.float32)]
            + [pltpu.SemaphoreType.DMA for _ in range(NBUF)]
        ),
        compiler_params=pltpu.CompilerParams(use_tc_tiling_on_sc=False),
    )
    def pool(idx_hbm, table_hbm, out_hbm, idx_v, *rest):
        bufs = rest[:NBUF]
        acc_v = rest[NBUF]
        sems = rest[NBUF + 1:]
        wid = lax.axis_index("s") * NC + lax.axis_index("c")
        pltpu.sync_copy(idx_hbm.at[wid], idx_v)

        zero = jnp.zeros((NL,), jnp.float32)

        def zbody(b, _):
            for g in range(G):
                acc_v[b, pl.ds(g * NL, NL)] = zero
            return 0

        lax.fori_loop(0, B_PER_W, zbody, 0)

        for k in range(NBUF):
            pltpu.async_copy(table_hbm.at[idx_v.at[k]], bufs[k], sems[k])

        n_iter = CPW // NBUF

        def outer(i, _):
            cbase = i * NBUF
            for k in range(NBUF):
                c = cbase + k
                pltpu.make_async_copy(
                    table_hbm.at[idx_v.at[c]], bufs[k], sems[k]).wait()
                buf = bufs[k]

                def red(j, carry):
                    s0 = j * UNROLL
                    out = list(carry)
                    for u in range(UNROLL):
                        for g in range(G):
                            out[g] = out[g] + buf[s0 + u, pl.ds(g * NL, NL)]
                    return tuple(out)

                sums = lax.fori_loop(0, REAL // UNROLL, red,
                                     tuple(zero for _ in range(G)))
                row = c // CPR
                for g in range(G):
                    plsc.addupdate(acc_v.at[row, pl.ds(g * NL, NL)],
                                   sums[g] * inv_s)

                @pl.when(i < n_iter - 1)
                def _():
                    pltpu.async_copy(
                        table_hbm.at[idx_v.at[c + NBUF]], bufs[k], sems[k])
            return 0

        lax.fori_loop(0, n_iter, outer, 0)
        pltpu.sync_copy(acc_v, out_hbm.at[pl.ds(wid * B_PER_W, B_PER_W)])

    return pool


def _fc_body(pooled_ref, w1t_ref, b1_ref, gamma_ref, beta_ref, rm_ref, rv_ref,
             w2t_ref, b2_ref, out_ref):
    h = jnp.dot(pooled_ref[...], w1t_ref[...],
                preferred_element_type=jnp.float32) + b1_ref[...]
    scale = gamma_ref[...] * lax.rsqrt(rv_ref[...] + EPS)
    h = (h - rm_ref[...]) * scale + beta_ref[...]
    h = jnp.maximum(h, 0.0)
    logits = jnp.dot(h, w2t_ref[...],
                     preferred_element_type=jnp.float32) + b2_ref[...]
    out_ref[...] = jax.nn.softmax(logits, axis=-1)


def _make_fc(B, D, H, LP, BB):
    grid = (B // BB,)
    full = lambda shape: pl.BlockSpec(shape, lambda i: (0,) * len(shape))
    return pl.pallas_call(
        _fc_body,
        grid=grid,
        in_specs=[
            pl.BlockSpec((BB, D), lambda i: (i, 0)),
            full((D, H)),
            full((1, H)),
            full((1, H)),
            full((1, H)),
            full((1, H)),
            full((1, H)),
            full((H, LP)),
            full((1, LP)),
        ],
        out_specs=pl.BlockSpec((BB, LP), lambda i: (i, 0)),
        out_shape=jax.ShapeDtypeStruct((B, LP), jnp.float32),
    )


def kernel(x, table, W1, b1, gamma, beta, rm, rv, W2, b2):
    B, S = x.shape
    V, D = table.shape
    H = W1.shape[0]
    L = W2.shape[0]

    REAL = S
    CHUNK = (REAL + 7) // 8 * 8

    idx = x.astype(jnp.int32).reshape(NW, B // NW, REAL)
    if CHUNK != REAL:
        idx = jnp.pad(idx, ((0, 0), (0, 0), (0, CHUNK - REAL)))

    table128 = jnp.pad(table, ((0, 0), (0, 128 - D)))
    pooled = _make_pool_kernel(B, S, D, 128, CHUNK, REAL)(idx, table128)

    LP = 128
    w2t = jnp.zeros((H, LP), jnp.float32).at[:, :L].set(W2.T)
    b2p = jnp.full((1, LP), -1e30, jnp.float32).at[0, :L].set(b2)

    probs = _make_fc(B, D, H, LP, 512)(
        pooled, W1.T, b1.reshape(1, H), gamma.reshape(1, H),
        beta.reshape(1, H), rm.reshape(1, H), rv.reshape(1, H), w2t, b2p)
    return probs[:, :L]


# speedup vs baseline: 1.1271x; 1.1271x over previous
import functools

import jax
import jax.numpy as jnp
from jax import lax
from jax.experimental import pallas as pl
from jax.experimental.pallas import tpu as pltpu
from jax.experimental.pallas import tpu_sc as plsc

EPS = 1e-5
DP = 128

_info = plsc.get_sparse_core_info()
NC, NS, NL = _info.num_cores, _info.num_subcores, _info.num_lanes
NW = NC * NS


def _transpose_body(eye_ref, tv_ref, out_ref):
    out_ref[...] = jax.lax.dot_general(
        tv_ref[...], eye_ref[...], (((0,), (0,)), ((), ())),
        preferred_element_type=jnp.float32,
        precision=jax.lax.Precision.HIGHEST)


def _make_transpose(V, D, BV):
    return pl.pallas_call(
        _transpose_body,
        grid=(pl.cdiv(V, BV),),
        in_specs=[
            pl.BlockSpec((D, DP), lambda i: (0, 0)),
            pl.BlockSpec((D, BV), lambda i: (0, i)),
        ],
        out_specs=pl.BlockSpec((BV, DP), lambda i: (i, 0)),
        out_shape=jax.ShapeDtypeStruct((V, DP), jnp.float32),
    )


def _make_pool_kernel(B, S, D):
    B_PER_W = B // NW
    G = D // NL
    GP = DP // NL
    inv_s = 1.0 / S
    NBUF = 3
    UNROLL = 4

    mesh = plsc.VectorSubcoreMesh(core_axis_name="c", subcore_axis_name="s")

    @functools.partial(
        pl.kernel,
        mesh=mesh,
        out_type=jax.ShapeDtypeStruct((B, DP), jnp.float32),
        scratch_types=(
            [pltpu.VMEM((B_PER_W * S,), jnp.int32)]
            + [pltpu.VMEM((S, DP), jnp.float32) for _ in range(NBUF)]
            + [pltpu.VMEM((B_PER_W, DP), jnp.float32)]
            + [pltpu.SemaphoreType.DMA for _ in range(NBUF)]
        ),
    )
    def pool(idx_hbm, table_hbm, out_hbm, idx_v, *rest):
        bufs = rest[:NBUF]
        acc_v = rest[NBUF]
        sems = rest[NBUF + 1:]
        wid = lax.axis_index("s") * NC + lax.axis_index("c")
        pltpu.sync_copy(idx_hbm.at[wid], idx_v)

        zero = jnp.zeros((NL,), jnp.float32)

        def zbody(b, _):
            for g in range(GP):
                acc_v[b, pl.ds(g * NL, NL)] = zero
            return 0

        lax.fori_loop(0, B_PER_W, zbody, 0)

        for k in range(NBUF):
            pltpu.async_copy(
                table_hbm.at[idx_v.at[pl.ds(k * S, S)]], bufs[k], sems[k])

        def outer(c, _):
            k = lax.rem(c, NBUF)
            for kk in range(NBUF):

                @pl.when(k == kk)
                def _():
                    pltpu.make_async_copy(
                        table_hbm.at[idx_v.at[pl.ds(c * S, S)]],
                        bufs[kk], sems[kk]).wait()
                    buf = bufs[kk]

                    def red(j, carry):
                        s0 = j * UNROLL
                        out = list(carry)
                        for u in range(UNROLL):
                            for g in range(G):
                                a = (u % 2) * G + g
                                out[a] = out[a] + buf[s0 + u, pl.ds(g * NL, NL)]
                        return tuple(out)

                    parts = lax.fori_loop(0, S // UNROLL, red,
                                          tuple(zero for _ in range(2 * G)))
                    for g in range(G):
                        acc_v[c, pl.ds(g * NL, NL)] = (
                            (parts[g] + parts[G + g]) * inv_s)

                    @pl.when(c < B_PER_W - NBUF)
                    def _():
                        pltpu.async_copy(
                            table_hbm.at[idx_v.at[pl.ds((c + NBUF) * S, S)]],
                            bufs[kk], sems[kk])
            return 0

        lax.fori_loop(0, B_PER_W, outer, 0)
        pltpu.sync_copy(acc_v, out_hbm.at[pl.ds(wid * B_PER_W, B_PER_W)])

    return pool


def _fc_body(pooled_ref, w1t_ref, b1_ref, gamma_ref, beta_ref, rm_ref, rv_ref,
             w2t_ref, b2_ref, out_ref):
    h = jnp.dot(pooled_ref[...], w1t_ref[...],
                preferred_element_type=jnp.float32) + b1_ref[...]
    scale = gamma_ref[...] * lax.rsqrt(rv_ref[...] + EPS)
    h = (h - rm_ref[...]) * scale + beta_ref[...]
    h = jnp.maximum(h, 0.0)
    logits = jnp.dot(h, w2t_ref[...],
                     preferred_element_type=jnp.float32) + b2_ref[...]
    out_ref[...] = jax.nn.softmax(logits, axis=-1)


def _make_fc(B, H, LP, BB):
    full = lambda shape: pl.BlockSpec(shape, lambda i: (0,) * len(shape))
    return pl.pallas_call(
        _fc_body,
        grid=(B // BB,),
        in_specs=[
            pl.BlockSpec((BB, DP), lambda i: (i, 0)),
            full((DP, H)),
            full((1, H)),
            full((1, H)),
            full((1, H)),
            full((1, H)),
            full((1, H)),
            full((H, LP)),
            full((1, LP)),
        ],
        out_specs=pl.BlockSpec((BB, LP), lambda i: (i, 0)),
        out_shape=jax.ShapeDtypeStruct((B, LP), jnp.float32),
    )


def kernel(x, table, W1, b1, gamma, beta, rm, rv, W2, b2):
    B, S = x.shape
    V, D = table.shape
    H = W1.shape[0]
    L = W2.shape[0]

    eye_pad = jnp.eye(D, DP, dtype=jnp.float32)
    table128 = _make_transpose(V, D, 6400)(eye_pad, table.T)

    idx = x.astype(jnp.int32).reshape(NW, (B // NW) * S)
    pooled = _make_pool_kernel(B, S, D)(idx, table128)

    LP = 128
    w1t = jnp.zeros((DP, H), jnp.float32).at[:D, :].set(W1.T)
    w2t = jnp.zeros((H, LP), jnp.float32).at[:, :L].set(W2.T)
    b2p = jnp.full((1, LP), -1e30, jnp.float32).at[0, :L].set(b2)

    probs = _make_fc(B, H, LP, 512)(
        pooled, w1t, b1.reshape(1, H), gamma.reshape(1, H),
        beta.reshape(1, H), rm.reshape(1, H), rv.reshape(1, H), w2t, b2p)
    return probs[:, :L]


# speedup vs baseline: 1.3822x; 1.2264x over previous
import functools

import jax
import jax.numpy as jnp
from jax import lax
from jax.experimental import pallas as pl
from jax.experimental.pallas import tpu as pltpu
from jax.experimental.pallas import tpu_sc as plsc

EPS = 1e-5
DP = 128

_info = plsc.get_sparse_core_info()
NC, NS, NL = _info.num_cores, _info.num_subcores, _info.num_lanes
NW = NC * NS


def _transpose_body(eye_ref, tv_ref, out_ref):
    x = tv_ref[...]
    h1 = x.astype(jnp.bfloat16)
    r1 = x - h1.astype(jnp.float32)
    h2 = r1.astype(jnp.bfloat16)
    r2 = r1 - h2.astype(jnp.float32)
    h3 = r2.astype(jnp.bfloat16)
    e = eye_ref[...]

    def t(p):
        return jax.lax.dot_general(
            p, e, (((0,), (0,)), ((), ())),
            preferred_element_type=jnp.float32)

    out_ref[...] = t(h1) + t(h2) + t(h3)


def _make_transpose(V, D, BV):
    return pl.pallas_call(
        _transpose_body,
        grid=(pl.cdiv(V, BV),),
        in_specs=[
            pl.BlockSpec((D, DP), lambda i: (0, 0)),
            pl.BlockSpec((D, BV), lambda i: (0, i)),
        ],
        out_specs=pl.BlockSpec((BV, DP), lambda i: (i, 0)),
        out_shape=jax.ShapeDtypeStruct((V, DP), jnp.float32),
        compiler_params=pltpu.CompilerParams(fuse_transposed_lhs_in_matmul=True),
    )


def _make_pool_kernel(B, S, D):
    B_PER_W = B // NW
    G = D // NL
    GP = DP // NL
    inv_s = 1.0 / S
    NBUF = 3
    UNROLL = 4

    mesh = plsc.VectorSubcoreMesh(core_axis_name="c", subcore_axis_name="s")

    @functools.partial(
        pl.kernel,
        mesh=mesh,
        out_type=jax.ShapeDtypeStruct((B, DP), jnp.float32),
        scratch_types=(
            [pltpu.VMEM((B_PER_W * S,), jnp.int32)]
            + [pltpu.VMEM((S, DP), jnp.float32) for _ in range(NBUF)]
            + [pltpu.VMEM((B_PER_W, DP), jnp.float32)]
            + [pltpu.SemaphoreType.DMA for _ in range(NBUF)]
        ),
    )
    def pool(idx_hbm, table_hbm, out_hbm, idx_v, *rest):
        bufs = rest[:NBUF]
        acc_v = rest[NBUF]
        sems = rest[NBUF + 1:]
        wid = lax.axis_index("s") * NC + lax.axis_index("c")
        pltpu.sync_copy(idx_hbm.at[wid], idx_v)

        zero = jnp.zeros((NL,), jnp.float32)

        def zbody(b, _):
            for g in range(GP):
                acc_v[b, pl.ds(g * NL, NL)] = zero
            return 0

        lax.fori_loop(0, B_PER_W, zbody, 0)

        for k in range(NBUF):
            pltpu.async_copy(
                table_hbm.at[idx_v.at[pl.ds(k * S, S)]], bufs[k], sems[k])

        def outer(c, _):
            k = lax.rem(c, NBUF)
            for kk in range(NBUF):

                @pl.when(k == kk)
                def _():
                    pltpu.make_async_copy(
                        table_hbm.at[idx_v.at[pl.ds(c * S, S)]],
                        bufs[kk], sems[kk]).wait()
                    buf = bufs[kk]

                    def red(j, carry):
                        s0 = j * UNROLL
                        out = list(carry)
                        for u in range(UNROLL):
                            for g in range(G):
                                a = (u % 2) * G + g
                                out[a] = out[a] + buf[s0 + u, pl.ds(g * NL, NL)]
                        return tuple(out)

                    parts = lax.fori_loop(0, S // UNROLL, red,
                                          tuple(zero for _ in range(2 * G)))
                    for g in range(G):
                        acc_v[c, pl.ds(g * NL, NL)] = (
                            (parts[g] + parts[G + g]) * inv_s)

                    @pl.when(c < B_PER_W - NBUF)
                    def _():
                        pltpu.async_copy(
                            table_hbm.at[idx_v.at[pl.ds((c + NBUF) * S, S)]],
                            bufs[kk], sems[kk])
            return 0

        lax.fori_loop(0, B_PER_W, outer, 0)
        pltpu.sync_copy(acc_v, out_hbm.at[pl.ds(wid * B_PER_W, B_PER_W)])

    return pool


def _fc_body(pooled_ref, w1t_ref, b1_ref, gamma_ref, beta_ref, rm_ref, rv_ref,
             w2t_ref, b2_ref, out_ref):
    h = jnp.dot(pooled_ref[...], w1t_ref[...],
                preferred_element_type=jnp.float32) + b1_ref[...]
    scale = gamma_ref[...] * lax.rsqrt(rv_ref[...] + EPS)
    h = (h - rm_ref[...]) * scale + beta_ref[...]
    h = jnp.maximum(h, 0.0)
    logits = jnp.dot(h, w2t_ref[...],
                     preferred_element_type=jnp.float32) + b2_ref[...]
    out_ref[...] = jax.nn.softmax(logits, axis=-1)


def _make_fc(B, H, LP, BB):
    full = lambda shape: pl.BlockSpec(shape, lambda i: (0,) * len(shape))
    return pl.pallas_call(
        _fc_body,
        grid=(B // BB,),
        in_specs=[
            pl.BlockSpec((BB, DP), lambda i: (i, 0)),
            full((DP, H)),
            full((1, H)),
            full((1, H)),
            full((1, H)),
            full((1, H)),
            full((1, H)),
            full((H, LP)),
            full((1, LP)),
        ],
        out_specs=pl.BlockSpec((BB, LP), lambda i: (i, 0)),
        out_shape=jax.ShapeDtypeStruct((B, LP), jnp.float32),
    )


def kernel(x, table, W1, b1, gamma, beta, rm, rv, W2, b2):
    B, S = x.shape
    V, D = table.shape
    H = W1.shape[0]
    L = W2.shape[0]

    eye_pad = jnp.eye(D, DP, dtype=jnp.bfloat16)
    table128 = _make_transpose(V, D, 6400)(eye_pad, table.T)

    idx = x.astype(jnp.int32).reshape(NW, (B // NW) * S)
    pooled = _make_pool_kernel(B, S, D)(idx, table128)

    LP = 128
    w1t = jnp.zeros((DP, H), jnp.float32).at[:D, :].set(W1.T)
    w2t = jnp.zeros((H, LP), jnp.float32).at[:, :L].set(W2.T)
    b2p = jnp.full((1, LP), -1e30, jnp.float32).at[0, :L].set(b2)

    probs = _make_fc(B, H, LP, 512)(
        pooled, w1t, b1.reshape(1, H), gamma.reshape(1, H),
        beta.reshape(1, H), rm.reshape(1, H), rv.reshape(1, H), w2t, b2p)
    return probs[:, :L]
